# baseline (speedup 1.0000x reference)
import jax
import jax.numpy as jnp
from jax import lax
from jax.experimental import pallas as pl
from jax.experimental.pallas import tpu as pltpu

_K = 8
_DVAR = 1.0
_DDIST = 2.0


def _body(lab_ref, x_ref, out_ref, acc_ref, mu_ref, loss_ref):
    phase = pl.program_id(0)
    c = pl.program_id(1)
    nc = pl.num_programs(1)
    x = x_ref[0]
    lab = lab_ref[0]
    cwidth = x.shape[1]
    n = cwidth * nc
    onehot = (lax.broadcasted_iota(jnp.int32, (_K, cwidth), 0) == lab).astype(
        jnp.float32)

    @pl.when((phase == 0) & (c == 0))
    def _init():
        acc_ref[...] = jnp.zeros_like(acc_ref)
        loss_ref[0, 0] = 0.0

    @pl.when(phase == 0)
    def _p0():
        acc_ref[...] += lax.dot_general(
            onehot, x, (((1,), (1,)), ((), ())),
            preferred_element_type=jnp.float32,
            precision=lax.Precision.HIGHEST)

    @pl.when((phase == 0) & (c == nc - 1))
    def _mid():
        mu = acc_ref[...] * (1.0 / n)
        mu_ref[...] = mu
        g = lax.dot_general(mu, mu, (((1,), (1,)), ((), ())),
                            preferred_element_type=jnp.float32,
                            precision=lax.Precision.HIGHEST)
        eye = (lax.broadcasted_iota(jnp.int32, (_K, _K), 0) ==
               lax.broadcasted_iota(jnp.int32, (_K, _K), 1)).astype(jnp.float32)
        dr = jnp.sum(g * eye, axis=1, keepdims=True)
        dc = jnp.sum(g * eye, axis=0, keepdims=True)
        d2 = jnp.maximum(dr + dc - 2.0 * g, 0.0)
        dist = jnp.maximum(_DDIST - jnp.sqrt(d2), 0.0) ** 2
        loss_ref[0, 0] = jnp.sum(dist) / (_K - 1) / 2.0

    @pl.when(phase == 1)
    def _p1():
        musel = lax.dot_general(
            mu_ref[...], onehot, (((0,), (0,)), ((), ())),
            preferred_element_type=jnp.float32,
            precision=lax.Precision.HIGHEST)
        diff = x - musel
        r2 = jnp.sum(diff * diff, axis=0, keepdims=True)
        t = jnp.maximum(jnp.sqrt(r2) - _DVAR, 0.0) ** 2
        loss_ref[0, 0] += jnp.sum(t) / n

    @pl.when((phase == 1) & (c == nc - 1))
    def _fin():
        out_ref[...] = jnp.full((1, 1), loss_ref[0, 0], jnp.float32)


def kernel(data, labels):
    b, d, h, w = data.shape
    n = h * w
    x = data.reshape(b, d, n)
    lab = labels.reshape(b, 1, n)
    nchunks = 8
    cwidth = n // nchunks
    out = pl.pallas_call(
        _body,
        grid=(2, nchunks),
        in_specs=[
            pl.BlockSpec((1, 1, cwidth), lambda p, c: (b - 1, 0, c)),
            pl.BlockSpec((1, d, cwidth), lambda p, c: (b - 1, 0, c)),
        ],
        out_specs=pl.BlockSpec((1, 1), lambda p, c: (0, 0)),
        out_shape=jax.ShapeDtypeStruct((1, 1), jnp.float32),
        scratch_shapes=[
            pltpu.VMEM((_K, d), jnp.float32),
            pltpu.VMEM((_K, d), jnp.float32),
            pltpu.SMEM((1, 1), jnp.float32),
        ],
        compiler_params=pltpu.CompilerParams(
            dimension_semantics=("arbitrary", "arbitrary")),
    )(lab, x)
    return out[0, 0]


# speedup vs baseline: 1.1625x; 1.1625x over previous
import jax
import jax.numpy as jnp
from jax import lax
from jax.experimental import pallas as pl
from jax.experimental.pallas import tpu as pltpu

_K = 8
_DVAR = 1.0
_DDIST = 2.0


def _body(lab_ref, x_ref, out_ref, acc_ref, mu_ref, loss_ref):
    phase = pl.program_id(0)
    c = pl.program_id(1)
    nc = pl.num_programs(1)
    x = x_ref[0]
    lab = lab_ref[0]
    cwidth = x.shape[1]
    n = cwidth * nc
    onehot = (lax.broadcasted_iota(jnp.int32, (_K, cwidth), 0) == lab).astype(
        jnp.float32)

    @pl.when((phase == 0) & (c == 0))
    def _init():
        acc_ref[...] = jnp.zeros_like(acc_ref)
        loss_ref[0, 0] = 0.0

    @pl.when(phase == 0)
    def _p0():
        acc_ref[...] += lax.dot_general(
            onehot, x, (((1,), (1,)), ((), ())),
            preferred_element_type=jnp.float32,
            precision=lax.Precision.DEFAULT)

    @pl.when((phase == 0) & (c == nc - 1))
    def _mid():
        mu = acc_ref[...] * (1.0 / n)
        mu_ref[...] = mu
        g = lax.dot_general(mu, mu, (((1,), (1,)), ((), ())),
                            preferred_element_type=jnp.float32,
                            precision=lax.Precision.DEFAULT)
        eye = (lax.broadcasted_iota(jnp.int32, (_K, _K), 0) ==
               lax.broadcasted_iota(jnp.int32, (_K, _K), 1)).astype(jnp.float32)
        dr = jnp.sum(g * eye, axis=1, keepdims=True)
        dc = jnp.sum(g * eye, axis=0, keepdims=True)
        d2 = jnp.maximum(dr + dc - 2.0 * g, 0.0)
        dist = jnp.maximum(_DDIST - jnp.sqrt(d2), 0.0) ** 2
        loss_ref[0, 0] = jnp.sum(dist) / (_K - 1) / 2.0

    @pl.when(phase == 1)
    def _p1():
        musel = lax.dot_general(
            mu_ref[...], onehot, (((0,), (0,)), ((), ())),
            preferred_element_type=jnp.float32,
            precision=lax.Precision.DEFAULT)
        diff = x - musel
        r2 = jnp.sum(diff * diff, axis=0, keepdims=True)
        t = jnp.maximum(jnp.sqrt(r2) - _DVAR, 0.0) ** 2
        loss_ref[0, 0] += jnp.sum(t) / n

    @pl.when((phase == 1) & (c == nc - 1))
    def _fin():
        out_ref[...] = jnp.full((1, 1), loss_ref[0, 0], jnp.float32)


def kernel(data, labels):
    b, d, h, w = data.shape
    n = h * w
    x = data.reshape(b, d, n)
    lab = labels.reshape(b, 1, n)
    nchunks = 8
    cwidth = n // nchunks
    out = pl.pallas_call(
        _body,
        grid=(2, nchunks),
        in_specs=[
            pl.BlockSpec((1, 1, cwidth), lambda p, c: (b - 1, 0, c)),
            pl.BlockSpec((1, d, cwidth), lambda p, c: (b - 1, 0, c)),
        ],
        out_specs=pl.BlockSpec((1, 1), lambda p, c: (0, 0)),
        out_shape=jax.ShapeDtypeStruct((1, 1), jnp.float32),
        scratch_shapes=[
            pltpu.VMEM((_K, d), jnp.float32),
            pltpu.VMEM((_K, d), jnp.float32),
            pltpu.SMEM((1, 1), jnp.float32),
        ],
        compiler_params=pltpu.CompilerParams(
            dimension_semantics=("arbitrary", "arbitrary")),
    )(lab, x)
    return out[0, 0]


# speedup vs baseline: 5.2711x; 4.5342x over previous
import jax
import jax.numpy as jnp
from jax import lax
from jax.experimental import pallas as pl
from jax.experimental.pallas import tpu as pltpu

_K = 8
_DVAR = 1.0
_DDIST = 2.0


def _body(lab_ref, x_ref, out_ref, acc_ref, mu_ref, loss_ref):
    phase = pl.program_id(0)
    c = pl.program_id(1)
    nc = pl.num_programs(1)
    x = x_ref[0]
    lab = lab_ref[0]
    d, hc, w = x.shape
    n = hc * w * nc
    onehot = (lax.broadcasted_iota(jnp.int32, (_K, hc, w), 0) ==
              lab[None]).astype(jnp.float32).reshape(_K, hc * w)
    x = x.reshape(d, hc * w)

    @pl.when((phase == 0) & (c == 0))
    def _init():
        acc_ref[...] = jnp.zeros_like(acc_ref)
        loss_ref[0, 0] = 0.0

    @pl.when(phase == 0)
    def _p0():
        acc_ref[...] += lax.dot_general(
            onehot, x, (((1,), (1,)), ((), ())),
            preferred_element_type=jnp.float32)

    @pl.when((phase == 0) & (c == nc - 1))
    def _mid():
        mu = acc_ref[...] * (1.0 / n)
        mu_ref[...] = mu
        g = lax.dot_general(mu, mu, (((1,), (1,)), ((), ())),
                            preferred_element_type=jnp.float32)
        eye = (lax.broadcasted_iota(jnp.int32, (_K, _K), 0) ==
               lax.broadcasted_iota(jnp.int32, (_K, _K), 1)).astype(jnp.float32)
        dr = jnp.sum(g * eye, axis=1, keepdims=True)
        dc = jnp.sum(g * eye, axis=0, keepdims=True)
        d2 = jnp.maximum(dr + dc - 2.0 * g, 0.0)
        dist = jnp.maximum(_DDIST - jnp.sqrt(d2), 0.0) ** 2
        loss_ref[0, 0] = jnp.sum(dist) / (_K - 1) / 2.0

    @pl.when(phase == 1)
    def _p1():
        musel = lax.dot_general(
            mu_ref[...], onehot, (((0,), (0,)), ((), ())),
            preferred_element_type=jnp.float32)
        diff = x - musel
        r2 = jnp.sum(diff * diff, axis=0, keepdims=True)
        t = jnp.maximum(jnp.sqrt(r2) - _DVAR, 0.0) ** 2
        loss_ref[0, 0] += jnp.sum(t) / n

    @pl.when((phase == 1) & (c == nc - 1))
    def _fin():
        out_ref[...] = jnp.full((1, 1), loss_ref[0, 0], jnp.float32)


def kernel(data, labels):
    b, d, h, w = data.shape
    nchunks = 8
    hc = h // nchunks
    out = pl.pallas_call(
        _body,
        grid=(2, nchunks),
        in_specs=[
            pl.BlockSpec((1, hc, w), lambda p, c: (b - 1, c, 0)),
            pl.BlockSpec((1, d, hc, w), lambda p, c: (b - 1, 0, c, 0)),
        ],
        out_specs=pl.BlockSpec((1, 1), lambda p, c: (0, 0)),
        out_shape=jax.ShapeDtypeStruct((1, 1), jnp.float32),
        scratch_shapes=[
            pltpu.VMEM((_K, d), jnp.float32),
            pltpu.VMEM((_K, d), jnp.float32),
            pltpu.SMEM((1, 1), jnp.float32),
        ],
        compiler_params=pltpu.CompilerParams(
            dimension_semantics=("arbitrary", "arbitrary")),
    )(labels, data)
    return out[0, 0]
